# Initial kernel scaffold; baseline (speedup 1.0000x reference)
import functools

import jax
import jax.numpy as jnp
from jax import lax
from jax.experimental import pallas as pl
from jax.experimental.pallas import tpu as pltpu
from jax.experimental.pallas import tpu_sc as plsc

N = 10000
NP = 10240
E = 320000
F_IN = 128
F = 64
NC, NS, L = 2, 16, 16
NW = NC * NS
CH = 128
EPT = -(-E // NW // CH) * CH
EP = EPT * NW
NPT = NP // NS
BN = 1024
NBLK = NP // BN

_f32 = jnp.float32


def _enc_body(x_ref, w_ref, asv_ref, adv_ref, h_ref, as_ref, ad_ref, sh_ref,
              m_ref):
    i = pl.program_id(0)
    h = jnp.dot(x_ref[...], w_ref[...], preferred_element_type=_f32)
    h_ref[...] = h
    asr = jnp.sum(h * asv_ref[...], axis=1, keepdims=True)
    adr = jnp.sum(h * adv_ref[...], axis=1, keepdims=True)
    as_ref[...] = asr
    ad_ref[...] = adr

    @pl.when(i == 0)
    def _():
        m_ref[0] = -jnp.inf
        m_ref[1] = -jnp.inf

    m_ref[0] = jnp.maximum(m_ref[0], jnp.max(asr))
    m_ref[1] = jnp.maximum(m_ref[1], jnp.max(adr))

    @pl.when(i == NBLK - 1)
    def _():
        g = m_ref[0] + m_ref[1]
        g = jnp.where(g >= 0.0, g, 0.2 * g)
        sh_ref[...] = jnp.full((8, 128), g, _f32)


def _encode(xp, W, att_src, att_dst):
    fin = xp.shape[1]
    return pl.pallas_call(
        _enc_body,
        grid=(NBLK,),
        in_specs=[
            pl.BlockSpec((BN, fin), lambda i: (i, 0)),
            pl.BlockSpec((fin, F), lambda i: (0, 0)),
            pl.BlockSpec((1, F), lambda i: (0, 0)),
            pl.BlockSpec((1, F), lambda i: (0, 0)),
        ],
        out_specs=[
            pl.BlockSpec((BN, F), lambda i: (i, 0)),
            pl.BlockSpec((BN, 1), lambda i: (i, 0)),
            pl.BlockSpec((BN, 1), lambda i: (i, 0)),
            pl.BlockSpec((8, 128), lambda i: (0, 0)),
        ],
        out_shape=[
            jax.ShapeDtypeStruct((NP, F), _f32),
            jax.ShapeDtypeStruct((NP, 1), _f32),
            jax.ShapeDtypeStruct((NP, 1), _f32),
            jax.ShapeDtypeStruct((8, 128), _f32),
        ],
        scratch_shapes=[pltpu.SMEM((2,), _f32)],
    )(xp, W, att_src, att_dst)


def _edge_body(h_hbm, asrc_hbm, adst_hbm, srcp_hbm, dstp_hbm, shift_hbm,
               acc_hbm, den_hbm,
               asrc_t, adst_t, sidx, didx, wbuf, rows, gbuf, zrow, zden,
               acc_s, den_s, gsem):
    cid = lax.axis_index("c")
    sid = lax.axis_index("s")
    w = cid * NS + sid

    zv = jnp.zeros((L,), _f32)

    def _zr(i, _):
        for j in range(F // L):
            zrow[i, pl.ds(j * L, L)] = zv
        return 0

    lax.fori_loop(0, CH, _zr, 0)

    def _zd(i, _):
        zden[pl.ds(i * L, L)] = zv
        return 0

    lax.fori_loop(0, NPT // L, _zd, 0)

    pltpu.sync_copy(asrc_hbm, asrc_t)
    pltpu.sync_copy(adst_hbm, adst_t)
    pltpu.sync_copy(shift_hbm.at[pl.ds(0, L)], gbuf)
    r0 = sid * NPT
    for r in range(NPT // CH):
        pltpu.sync_copy(zrow, acc_s.at[pl.ds(r0 + r * CH, CH)])
    pltpu.sync_copy(zden, den_s.at[pl.ds(r0, NPT)])
    plsc.subcore_barrier()

    gv = gbuf[...]
    base0 = w * EPT

    def _chunk(c, _):
        b = base0 + c * CH
        pltpu.sync_copy(srcp_hbm.at[pl.ds(b, CH)], sidx)
        pltpu.sync_copy(dstp_hbm.at[pl.ds(b, CH)], didx)
        cp = pltpu.async_copy(h_hbm.at[sidx], rows, gsem)

        def _grp(i, _):
            sv = sidx[pl.ds(i * L, L)]
            dv = didx[pl.ds(i * L, L)]
            ag = plsc.load_gather(asrc_t, [sv])
            dg = plsc.load_gather(adst_t, [dv])
            al = ag + dg
            al = jnp.where(al >= 0.0, al, 0.2 * al)
            wbuf[pl.ds(i * L, L)] = jnp.exp(al - gv)
            return 0

        lax.fori_loop(0, CH // L, _grp, 0)
        cp.wait()

        def _scale(e, _):
            wv = plsc.load_gather(wbuf, [jnp.full((L,), e, jnp.int32)])
            for j in range(F // L):
                rows[e, pl.ds(j * L, L)] = rows[e, pl.ds(j * L, L)] * wv
            return 0

        lax.fori_loop(0, CH, _scale, 0)
        pltpu.sync_copy(rows, acc_s.at[didx], add=True)
        pltpu.sync_copy(wbuf, den_s.at[didx], add=True)
        return 0

    lax.fori_loop(0, EPT // CH, _chunk, 0)
    plsc.subcore_barrier()

    pltpu.sync_copy(acc_s.at[pl.ds(r0, NPT)], acc_hbm.at[cid, pl.ds(r0, NPT)])
    pltpu.sync_copy(den_s.at[pl.ds(r0, NPT)], den_hbm.at[cid, pl.ds(r0, NPT)])


_edge = pl.kernel(
    _edge_body,
    out_type=[
        jax.ShapeDtypeStruct((NC, NP, F), _f32),
        jax.ShapeDtypeStruct((NC, NP), _f32),
    ],
    mesh=plsc.VectorSubcoreMesh(core_axis_name="c", subcore_axis_name="s",
                                num_cores=NC, num_subcores=NS),
    scratch_types=[
        pltpu.VMEM((NP,), _f32),
        pltpu.VMEM((NP,), _f32),
        pltpu.VMEM((CH,), jnp.int32),
        pltpu.VMEM((CH,), jnp.int32),
        pltpu.VMEM((CH,), _f32),
        pltpu.VMEM((CH, F), _f32),
        pltpu.VMEM((L,), _f32),
        pltpu.VMEM((CH, F), _f32),
        pltpu.VMEM((NPT,), _f32),
        pltpu.VMEM_SHARED((NP, F), _f32),
        pltpu.VMEM_SHARED((NP,), _f32),
        pltpu.SemaphoreType.DMA,
    ],
)


def _comb_body(acc_ref, den_ref, h_ref, as_ref, ad_ref, sh_ref, b_ref, w_ref,
               asv_ref, adv_ref, hw_ref, as2_ref, ad2_ref, sh2_ref, m_ref):
    i = pl.program_id(0)
    al = as_ref[...] + ad_ref[...]
    al = jnp.where(al >= 0.0, al, 0.2 * al)
    ws = jnp.exp(al - jnp.max(sh_ref[...]))
    tot = acc_ref[0] + acc_ref[1] + ws * h_ref[...]
    dtot = den_ref[:, 0:1] + den_ref[:, 1:2] + ws
    h2 = jnp.maximum(tot / (dtot + 1e-16) + b_ref[...], 0.0)
    hw = jnp.dot(h2, w_ref[...], preferred_element_type=_f32)
    hw_ref[...] = hw
    asr = jnp.sum(hw * asv_ref[...], axis=1, keepdims=True)
    adr = jnp.sum(hw * adv_ref[...], axis=1, keepdims=True)
    as2_ref[...] = asr
    ad2_ref[...] = adr

    @pl.when(i == 0)
    def _():
        m_ref[0] = -jnp.inf
        m_ref[1] = -jnp.inf

    m_ref[0] = jnp.maximum(m_ref[0], jnp.max(asr))
    m_ref[1] = jnp.maximum(m_ref[1], jnp.max(adr))

    @pl.when(i == NBLK - 1)
    def _():
        g = m_ref[0] + m_ref[1]
        g = jnp.where(g >= 0.0, g, 0.2 * g)
        sh2_ref[...] = jnp.full((8, 128), g, _f32)


def _combine(acc, dent, h, a_s, a_d, sh, b, W, att_src, att_dst):
    return pl.pallas_call(
        _comb_body,
        grid=(NBLK,),
        in_specs=[
            pl.BlockSpec((NC, BN, F), lambda i: (0, i, 0)),
            pl.BlockSpec((BN, NC), lambda i: (i, 0)),
            pl.BlockSpec((BN, F), lambda i: (i, 0)),
            pl.BlockSpec((BN, 1), lambda i: (i, 0)),
            pl.BlockSpec((BN, 1), lambda i: (i, 0)),
            pl.BlockSpec((8, 128), lambda i: (0, 0)),
            pl.BlockSpec((1, F), lambda i: (0, 0)),
            pl.BlockSpec((F, F), lambda i: (0, 0)),
            pl.BlockSpec((1, F), lambda i: (0, 0)),
            pl.BlockSpec((1, F), lambda i: (0, 0)),
        ],
        out_specs=[
            pl.BlockSpec((BN, F), lambda i: (i, 0)),
            pl.BlockSpec((BN, 1), lambda i: (i, 0)),
            pl.BlockSpec((BN, 1), lambda i: (i, 0)),
            pl.BlockSpec((8, 128), lambda i: (0, 0)),
        ],
        out_shape=[
            jax.ShapeDtypeStruct((NP, F), _f32),
            jax.ShapeDtypeStruct((NP, 1), _f32),
            jax.ShapeDtypeStruct((NP, 1), _f32),
            jax.ShapeDtypeStruct((8, 128), _f32),
        ],
        scratch_shapes=[pltpu.SMEM((2,), _f32)],
    )(acc, dent, h, a_s, a_d, sh, b, W, att_src, att_dst)


def _fin_body(acc_ref, den_ref, h_ref, as_ref, ad_ref, sh_ref, b_ref,
              out_ref):
    al = as_ref[...] + ad_ref[...]
    al = jnp.where(al >= 0.0, al, 0.2 * al)
    ws = jnp.exp(al - jnp.max(sh_ref[...]))
    tot = acc_ref[0] + acc_ref[1] + ws * h_ref[...]
    dtot = den_ref[:, 0:1] + den_ref[:, 1:2] + ws
    out_ref[...] = tot / (dtot + 1e-16) + b_ref[...]


def _final(acc, dent, h, a_s, a_d, sh, b):
    return pl.pallas_call(
        _fin_body,
        grid=(NBLK,),
        in_specs=[
            pl.BlockSpec((NC, BN, F), lambda i: (0, i, 0)),
            pl.BlockSpec((BN, NC), lambda i: (i, 0)),
            pl.BlockSpec((BN, F), lambda i: (i, 0)),
            pl.BlockSpec((BN, 1), lambda i: (i, 0)),
            pl.BlockSpec((BN, 1), lambda i: (i, 0)),
            pl.BlockSpec((8, 128), lambda i: (0, 0)),
            pl.BlockSpec((1, F), lambda i: (0, 0)),
        ],
        out_specs=[pl.BlockSpec((BN, F), lambda i: (i, 0))],
        out_shape=jax.ShapeDtypeStruct((NP, F), _f32),
    )(acc, dent, h, a_s, a_d, sh, b)


def kernel(x, edge_index, W1, att_src1, att_dst1, b1, W2, att_src2, att_dst2,
           b2):
    xp = jnp.zeros((NP, F_IN), _f32).at[:N].set(x)
    pidx = jnp.arange(EP - E, dtype=jnp.int32)
    srcp = jnp.concatenate([edge_index[0], pidx % N])
    dstp = jnp.concatenate([edge_index[1], N + pidx % (NP - N)])

    h1, as1, ad1, sh1 = _encode(xp, W1, att_src1, att_dst1)
    acc1, den1 = _edge(h1, as1.reshape(NP), ad1.reshape(NP), srcp, dstp,
                       sh1.reshape(8 * 128))
    h2w, as2, ad2, sh2 = _combine(acc1, den1.T, h1, as1, ad1, sh1,
                                  b1.reshape(1, F), W2, att_src2, att_dst2)
    acc2, den2 = _edge(h2w, as2.reshape(NP), ad2.reshape(NP), srcp, dstp,
                       sh2.reshape(8 * 128))
    outp = _final(acc2, den2.T, h2w, as2, ad2, sh2, b2.reshape(1, F))
    return outp[:N]


# speedup vs baseline: 27.9721x; 27.9721x over previous
import functools

import jax
import jax.numpy as jnp
from jax import lax
from jax.experimental import pallas as pl
from jax.experimental.pallas import tpu as pltpu
from jax.experimental.pallas import tpu_sc as plsc

N = 10000
NP = 10240
E = 320000
F_IN = 128
F = 64
FW = 128
NC, NS, L = 2, 16, 16
NW = NC * NS
CH = 128
EPT = -(-E // NW // CH) * CH
EP = EPT * NW
NPT = NP // NS
NSCALE = F // L + 1
BN = 1024
NBLK = NP // BN

_f32 = jnp.float32


def _table_row(h):
    n = h.shape[0]
    return jnp.concatenate(
        [h, jnp.full((n, 1), 1.0, _f32), jnp.zeros((n, FW - F - 1), _f32)],
        axis=1)


def _enc_body(x_ref, w_ref, asv_ref, adv_ref, t_ref, as_ref, ad_ref, sh_ref,
              m_ref):
    i = pl.program_id(0)
    h = jnp.dot(x_ref[...], w_ref[...], preferred_element_type=_f32)
    t_ref[...] = _table_row(h)
    asr = jnp.sum(h * asv_ref[...], axis=1, keepdims=True)
    adr = jnp.sum(h * adv_ref[...], axis=1, keepdims=True)
    as_ref[...] = asr
    ad_ref[...] = adr

    @pl.when(i == 0)
    def _():
        m_ref[0] = -jnp.inf
        m_ref[1] = -jnp.inf

    m_ref[0] = jnp.maximum(m_ref[0], jnp.max(asr))
    m_ref[1] = jnp.maximum(m_ref[1], jnp.max(adr))

    @pl.when(i == NBLK - 1)
    def _():
        g = m_ref[0] + m_ref[1]
        g = jnp.where(g >= 0.0, g, 0.2 * g)
        sh_ref[...] = jnp.full((8, 128), g, _f32)


def _encode(xp, W, att_src, att_dst):
    fin = xp.shape[1]
    return pl.pallas_call(
        _enc_body,
        grid=(NBLK,),
        in_specs=[
            pl.BlockSpec((BN, fin), lambda i: (i, 0)),
            pl.BlockSpec((fin, F), lambda i: (0, 0)),
            pl.BlockSpec((1, F), lambda i: (0, 0)),
            pl.BlockSpec((1, F), lambda i: (0, 0)),
        ],
        out_specs=[
            pl.BlockSpec((BN, FW), lambda i: (i, 0)),
            pl.BlockSpec((BN, 1), lambda i: (i, 0)),
            pl.BlockSpec((BN, 1), lambda i: (i, 0)),
            pl.BlockSpec((8, 128), lambda i: (0, 0)),
        ],
        out_shape=[
            jax.ShapeDtypeStruct((NP, FW), _f32),
            jax.ShapeDtypeStruct((NP, 1), _f32),
            jax.ShapeDtypeStruct((NP, 1), _f32),
            jax.ShapeDtypeStruct((8, 128), _f32),
        ],
        scratch_shapes=[pltpu.SMEM((2,), _f32)],
    )(xp, W, att_src, att_dst)


def _edge_body(tab_hbm, asrc_hbm, adst_hbm, srcp_hbm, dstp_hbm, shift_hbm,
               acc_hbm,
               asrc_t, adst_t, sidx, didx, wbuf, rows, gbuf,
               acc_s, gsem):
    cid = lax.axis_index("c")
    sid = lax.axis_index("s")
    w = cid * NS + sid

    zv = jnp.zeros((L,), _f32)

    def _zr(i, _):
        for j in range(FW // L):
            rows[i, pl.ds(j * L, L)] = zv
        return 0

    lax.fori_loop(0, CH, _zr, 0)

    pltpu.sync_copy(asrc_hbm, asrc_t)
    pltpu.sync_copy(adst_hbm, adst_t)
    pltpu.sync_copy(shift_hbm.at[pl.ds(0, L)], gbuf)
    r0 = sid * NPT
    for r in range(NPT // CH):
        pltpu.sync_copy(rows, acc_s.at[pl.ds(r0 + r * CH, CH)])
    plsc.subcore_barrier()

    gv = gbuf[...]
    base0 = w * EPT

    def _chunk(c, _):
        b = base0 + c * CH
        pltpu.sync_copy(srcp_hbm.at[pl.ds(b, CH)], sidx)
        pltpu.sync_copy(dstp_hbm.at[pl.ds(b, CH)], didx)
        cp = pltpu.async_copy(tab_hbm.at[sidx], rows, gsem)

        def _grp(i, _):
            sv = sidx[pl.ds(i * L, L)]
            dv = didx[pl.ds(i * L, L)]
            ag = plsc.load_gather(asrc_t, [sv])
            dg = plsc.load_gather(adst_t, [dv])
            al = ag + dg
            al = jnp.where(al >= 0.0, al, 0.2 * al)
            wbuf[pl.ds(i * L, L)] = jnp.exp(al - gv)
            return 0

        lax.fori_loop(0, CH // L, _grp, 0)
        cp.wait()

        def _scale(e, _):
            wv = plsc.load_gather(wbuf, [jnp.full((L,), e, jnp.int32)])
            for j in range(NSCALE):
                rows[e, pl.ds(j * L, L)] = rows[e, pl.ds(j * L, L)] * wv
            return 0

        lax.fori_loop(0, CH, _scale, 0)
        pltpu.sync_copy(rows, acc_s.at[didx], add=True)
        return 0

    lax.fori_loop(0, EPT // CH, _chunk, 0)
    plsc.subcore_barrier()

    pltpu.sync_copy(acc_s.at[pl.ds(r0, NPT)], acc_hbm.at[cid, pl.ds(r0, NPT)])


@functools.lru_cache(maxsize=None)
def _edge_kernel():
    return pl.kernel(
        _edge_body,
        out_type=[jax.ShapeDtypeStruct((NC, NP, FW), _f32)],
        mesh=plsc.VectorSubcoreMesh(core_axis_name="c", subcore_axis_name="s",
                                    num_cores=NC, num_subcores=NS),
        scratch_types=[
            pltpu.VMEM((NP,), _f32),
            pltpu.VMEM((NP,), _f32),
            pltpu.VMEM((CH,), jnp.int32),
            pltpu.VMEM((CH,), jnp.int32),
            pltpu.VMEM((CH,), _f32),
            pltpu.VMEM((CH, FW), _f32),
            pltpu.VMEM((L,), _f32),
            pltpu.VMEM_SHARED((NP, FW), _f32),
            pltpu.SemaphoreType.DMA,
        ],
        compiler_params=pltpu.CompilerParams(needs_layout_passes=False),
    )


def _edge(*args):
    return _edge_kernel()(*args)[0]


def _comb_body(acc_ref, t_ref, as_ref, ad_ref, sh_ref, b_ref, w_ref,
               asv_ref, adv_ref, t2_ref, as2_ref, ad2_ref, sh2_ref, m_ref):
    i = pl.program_id(0)
    al = as_ref[...] + ad_ref[...]
    al = jnp.where(al >= 0.0, al, 0.2 * al)
    ws = jnp.exp(al - jnp.max(sh_ref[...]))
    tot = acc_ref[0] + acc_ref[1] + ws * t_ref[...]
    h2 = jnp.maximum(tot[:, :F] / (tot[:, F:F + 1] + 1e-16) + b_ref[...], 0.0)
    hw = jnp.dot(h2, w_ref[...], preferred_element_type=_f32)
    t2_ref[...] = _table_row(hw)
    asr = jnp.sum(hw * asv_ref[...], axis=1, keepdims=True)
    adr = jnp.sum(hw * adv_ref[...], axis=1, keepdims=True)
    as2_ref[...] = asr
    ad2_ref[...] = adr

    @pl.when(i == 0)
    def _():
        m_ref[0] = -jnp.inf
        m_ref[1] = -jnp.inf

    m_ref[0] = jnp.maximum(m_ref[0], jnp.max(asr))
    m_ref[1] = jnp.maximum(m_ref[1], jnp.max(adr))

    @pl.when(i == NBLK - 1)
    def _():
        g = m_ref[0] + m_ref[1]
        g = jnp.where(g >= 0.0, g, 0.2 * g)
        sh2_ref[...] = jnp.full((8, 128), g, _f32)


def _combine(acc, tab, a_s, a_d, sh, b, W, att_src, att_dst):
    return pl.pallas_call(
        _comb_body,
        grid=(NBLK,),
        in_specs=[
            pl.BlockSpec((NC, BN, FW), lambda i: (0, i, 0)),
            pl.BlockSpec((BN, FW), lambda i: (i, 0)),
            pl.BlockSpec((BN, 1), lambda i: (i, 0)),
            pl.BlockSpec((BN, 1), lambda i: (i, 0)),
            pl.BlockSpec((8, 128), lambda i: (0, 0)),
            pl.BlockSpec((1, F), lambda i: (0, 0)),
            pl.BlockSpec((F, F), lambda i: (0, 0)),
            pl.BlockSpec((1, F), lambda i: (0, 0)),
            pl.BlockSpec((1, F), lambda i: (0, 0)),
        ],
        out_specs=[
            pl.BlockSpec((BN, FW), lambda i: (i, 0)),
            pl.BlockSpec((BN, 1), lambda i: (i, 0)),
            pl.BlockSpec((BN, 1), lambda i: (i, 0)),
            pl.BlockSpec((8, 128), lambda i: (0, 0)),
        ],
        out_shape=[
            jax.ShapeDtypeStruct((NP, FW), _f32),
            jax.ShapeDtypeStruct((NP, 1), _f32),
            jax.ShapeDtypeStruct((NP, 1), _f32),
            jax.ShapeDtypeStruct((8, 128), _f32),
        ],
        scratch_shapes=[pltpu.SMEM((2,), _f32)],
    )(acc, tab, a_s, a_d, sh, b, W, att_src, att_dst)


def _fin_body(acc_ref, t_ref, as_ref, ad_ref, sh_ref, b_ref, out_ref):
    al = as_ref[...] + ad_ref[...]
    al = jnp.where(al >= 0.0, al, 0.2 * al)
    ws = jnp.exp(al - jnp.max(sh_ref[...]))
    tot = acc_ref[0] + acc_ref[1] + ws * t_ref[...]
    out_ref[...] = tot[:, :F] / (tot[:, F:F + 1] + 1e-16) + b_ref[...]


def _final(acc, tab, a_s, a_d, sh, b):
    return pl.pallas_call(
        _fin_body,
        grid=(NBLK,),
        in_specs=[
            pl.BlockSpec((NC, BN, FW), lambda i: (0, i, 0)),
            pl.BlockSpec((BN, FW), lambda i: (i, 0)),
            pl.BlockSpec((BN, 1), lambda i: (i, 0)),
            pl.BlockSpec((BN, 1), lambda i: (i, 0)),
            pl.BlockSpec((8, 128), lambda i: (0, 0)),
            pl.BlockSpec((1, F), lambda i: (0, 0)),
        ],
        out_specs=pl.BlockSpec((BN, F), lambda i: (i, 0)),
        out_shape=jax.ShapeDtypeStruct((NP, F), _f32),
    )(acc, tab, a_s, a_d, sh, b)


def kernel(x, edge_index, W1, att_src1, att_dst1, b1, W2, att_src2, att_dst2,
           b2):
    xp = jnp.zeros((NP, F_IN), _f32).at[:N].set(x)
    pidx = jnp.arange(EP - E, dtype=jnp.int32)
    srcp = jnp.concatenate([edge_index[0], pidx % N])
    dstp = jnp.concatenate([edge_index[1], N + pidx % (NP - N)])

    tab1, as1, ad1, sh1 = _encode(xp, W1, att_src1, att_dst1)
    acc1 = _edge(tab1, as1.reshape(NP), ad1.reshape(NP), srcp, dstp,
                 sh1.reshape(8 * 128))
    tab2, as2, ad2, sh2 = _combine(acc1, tab1, as1, ad1, sh1,
                                   b1.reshape(1, F), W2, att_src2, att_dst2)
    acc2 = _edge(tab2, as2.reshape(NP), ad2.reshape(NP), srcp, dstp,
                 sh2.reshape(8 * 128))
    outp = _final(acc2, tab2, as2, ad2, sh2, b2.reshape(1, F))
    return outp[:N]
